# baseline (speedup 1.0000x reference)
import functools

import jax
import jax.numpy as jnp
from jax import lax
from jax.experimental import pallas as pl
from jax.experimental.pallas import tpu as pltpu
from jax.experimental.pallas import tpu_sc as plsc

_C = 8
_NBUF = 8


@functools.cache
def _sc_call(N, D, V):
    info = plsc.get_sparse_core_info()
    nw = info.num_cores * info.num_subcores
    rows_w = N // nw
    n_chunks = rows_w // _C
    assert n_chunks % _NBUF == 0 and n_chunks >= 2 * _NBUF

    mesh = plsc.VectorSubcoreMesh(core_axis_name="c", subcore_axis_name="s")

    @functools.partial(
        pl.kernel,
        mesh=mesh,
        compiler_params=pltpu.CompilerParams(needs_layout_passes=False),
        out_type=jax.ShapeDtypeStruct((N, D), jnp.float32),
        scratch_types=[
            pltpu.VMEM((rows_w,), jnp.int32),
            pltpu.VMEM((_NBUF, _C, D), jnp.float32),
            pltpu.VMEM((_NBUF, _C, D // 2), jnp.int32),
        ]
        + [pltpu.SemaphoreType.DMA] * (3 * _NBUF),
    )
    def k(x_hbm, idx_hbm, table_hbm, out_hbm, idx_v, x_bufs, emb_bufs, *sems):
        in_x_sems = sems[0:_NBUF]
        in_e_sems = sems[_NBUF:2 * _NBUF]
        out_sems = sems[2 * _NBUF:3 * _NBUF]
        wid = lax.axis_index("s") * info.num_cores + lax.axis_index("c")
        base = wid * rows_w
        pltpu.sync_copy(idx_hbm.at[pl.ds(base, rows_w)], idx_v)

        def issue_in(ci, b):
            r0 = base + ci * _C
            pltpu.async_copy(x_hbm.at[pl.ds(r0, _C)], x_bufs.at[b], in_x_sems[b])
            pltpu.async_copy(
                table_hbm.at[idx_v.at[pl.ds(ci * _C, _C)]],
                emb_bufs.at[b],
                in_e_sems[b],
            )

        def wait_in(ci, b):
            pltpu.make_async_copy(
                x_hbm.at[pl.ds(base, _C)], x_bufs.at[b], in_x_sems[b]
            ).wait()
            pltpu.make_async_copy(
                table_hbm.at[idx_v.at[pl.ds(ci * _C, _C)]],
                emb_bufs.at[b],
                in_e_sems[b],
            ).wait()

        def wait_out(b):
            pltpu.make_async_copy(
                x_bufs.at[b], out_hbm.at[pl.ds(base, _C)], out_sems[b]
            ).wait()

        hi_mask = jnp.full((16,), -65536, jnp.int32)

        def compute(b):
            def row_body(r, rcarry):
                for c in range(D // 32):
                    w = emb_bufs[b, r, pl.ds(c * 16, 16)]
                    lo = plsc.bitcast(w << 16, jnp.float32)
                    hi = plsc.bitcast(w & hi_mask, jnp.float32)
                    sl0 = pl.ds(c * 32, 16)
                    sl1 = pl.ds(c * 32 + 16, 16)
                    x_bufs[b, r, sl0] = x_bufs[b, r, sl0] + lo
                    x_bufs[b, r, sl1] = x_bufs[b, r, sl1] + hi
                return rcarry

            lax.fori_loop(0, _C, row_body, 0, unroll=False)

        for b in range(_NBUF - 1):
            issue_in(b, b)

        def step(ci, b):
            b_next = (b + _NBUF - 1) % _NBUF

            @pl.when(ci + _NBUF - 1 < n_chunks)
            def _():
                @pl.when(ci >= 1)
                def _():
                    wait_out(b_next)

                issue_in(ci + _NBUF - 1, b_next)

            wait_in(ci, b)
            compute(b)
            r0 = base + ci * _C
            pltpu.async_copy(x_bufs.at[b], out_hbm.at[pl.ds(r0, _C)], out_sems[b])

        def group_body(g, carry):
            for b in range(_NBUF):
                step(_NBUF * g + b, b)
            return carry

        lax.fori_loop(0, n_chunks // _NBUF, group_body, 0, unroll=False)
        for b in range(_NBUF):
            wait_out(b)

    return k


def _pack_table(table):
    V, D = table.shape
    bits = lax.bitcast_convert_type(table.astype(jnp.bfloat16), jnp.uint16)
    r = bits.reshape(V, D // 32, 2, 16).astype(jnp.uint32)
    w = r[:, :, 0, :] | (r[:, :, 1, :] << 16)
    return lax.bitcast_convert_type(w.reshape(V, D // 2), jnp.int32)


def kernel(x, indices, table):
    B, S, D = x.shape
    V = table.shape[0]
    N = B * S
    x2 = x.reshape(N, D)
    idx2 = indices.reshape(N).astype(jnp.int32)
    out = _sc_call(N, D, V)(x2, idx2, _pack_table(table))
    return out.reshape(B, S, D)


# speedup vs baseline: 1.0224x; 1.0224x over previous
import functools

import jax
import jax.numpy as jnp
from jax import lax
from jax.experimental import pallas as pl
from jax.experimental.pallas import tpu as pltpu
from jax.experimental.pallas import tpu_sc as plsc

_C = 8
_NBUF = 4


@functools.cache
def _sc_call(N, D, V):
    info = plsc.get_sparse_core_info()
    nw = info.num_cores * info.num_subcores
    rows_w = N // nw
    n_chunks = rows_w // _C
    assert n_chunks % _NBUF == 0 and n_chunks >= 2 * _NBUF

    mesh = plsc.VectorSubcoreMesh(core_axis_name="c", subcore_axis_name="s")

    @functools.partial(
        pl.kernel,
        mesh=mesh,
        compiler_params=pltpu.CompilerParams(needs_layout_passes=False),
        out_type=jax.ShapeDtypeStruct((N, D), jnp.float32),
        scratch_types=[
            pltpu.VMEM((rows_w,), jnp.int32),
            pltpu.VMEM((_NBUF, _C, D), jnp.float32),
            pltpu.VMEM((_NBUF, _C, D // 2), jnp.int32),
        ]
        + [pltpu.SemaphoreType.DMA] * (3 * _NBUF),
    )
    def k(x_hbm, idx_hbm, table_hbm, out_hbm, idx_v, x_bufs, emb_bufs, *sems):
        in_x_sems = sems[0:_NBUF]
        in_e_sems = sems[_NBUF:2 * _NBUF]
        out_sems = sems[2 * _NBUF:3 * _NBUF]
        wid = lax.axis_index("s") * info.num_cores + lax.axis_index("c")
        base = wid * rows_w
        pltpu.sync_copy(idx_hbm.at[pl.ds(base, rows_w)], idx_v)

        def issue_in(ci, b):
            r0 = base + ci * _C
            pltpu.async_copy(x_hbm.at[pl.ds(r0, _C)], x_bufs.at[b], in_x_sems[b])
            pltpu.async_copy(
                table_hbm.at[idx_v.at[pl.ds(ci * _C, _C)]],
                emb_bufs.at[b],
                in_e_sems[b],
            )

        def wait_in(ci, b):
            pltpu.make_async_copy(
                x_hbm.at[pl.ds(base, _C)], x_bufs.at[b], in_x_sems[b]
            ).wait()
            pltpu.make_async_copy(
                table_hbm.at[idx_v.at[pl.ds(ci * _C, _C)]],
                emb_bufs.at[b],
                in_e_sems[b],
            ).wait()

        def wait_out(b):
            pltpu.make_async_copy(
                x_bufs.at[b], out_hbm.at[pl.ds(base, _C)], out_sems[b]
            ).wait()

        hi_mask = jnp.full((16,), -65536, jnp.int32)

        def compute(b):
            def row_body(r, rcarry):
                for c in range(D // 32):
                    w = emb_bufs[b, r, pl.ds(c * 16, 16)]
                    lo = plsc.bitcast(w << 16, jnp.float32)
                    hi = plsc.bitcast(w & hi_mask, jnp.float32)
                    sl0 = pl.ds(c * 32, 16)
                    sl1 = pl.ds(c * 32 + 16, 16)
                    x_bufs[b, r, sl0] = x_bufs[b, r, sl0] + lo
                    x_bufs[b, r, sl1] = x_bufs[b, r, sl1] + hi
                return rcarry

            lax.fori_loop(0, _C, row_body, 0, unroll=False)

        for b in range(_NBUF - 1):
            issue_in(b, b)

        def step(ci, b):
            b_next = (b + _NBUF - 1) % _NBUF

            @pl.when(ci + _NBUF - 1 < n_chunks)
            def _():
                @pl.when(ci >= 1)
                def _():
                    wait_out(b_next)

                issue_in(ci + _NBUF - 1, b_next)

            wait_in(ci, b)
            compute(b)
            r0 = base + ci * _C
            pltpu.async_copy(x_bufs.at[b], out_hbm.at[pl.ds(r0, _C)], out_sems[b])

        def group_body(g, carry):
            for b in range(_NBUF):
                step(_NBUF * g + b, b)
            return carry

        lax.fori_loop(0, n_chunks // _NBUF, group_body, 0, unroll=False)
        for b in range(_NBUF):
            wait_out(b)

    return k


def _pack_table(table):
    V, D = table.shape
    bits = lax.bitcast_convert_type(table.astype(jnp.bfloat16), jnp.uint16)
    r = bits.reshape(V, D // 32, 2, 16).astype(jnp.uint32)
    w = r[:, :, 0, :] | (r[:, :, 1, :] << 16)
    return lax.bitcast_convert_type(w.reshape(V, D // 2), jnp.int32)


def kernel(x, indices, table):
    B, S, D = x.shape
    V = table.shape[0]
    N = B * S
    x2 = x.reshape(N, D)
    idx2 = indices.reshape(N).astype(jnp.int32)
    out = _sc_call(N, D, V)(x2, idx2, _pack_table(table))
    return out.reshape(B, S, D)
